# baseline (speedup 1.0000x reference)
import functools

import jax
import jax.numpy as jnp
from jax import lax
from jax.experimental import pallas as pl
from jax.experimental.pallas import tpu as pltpu
from jax.experimental.pallas import tpu_sc as plsc

N_NODES = 10000
N_EDGES = 320000
D = 128

SPLITS = ((0, 76800), (76800, 76800), (153600, 76800), (230400, 89600))
NSPLIT = len(SPLITS)

NC = 2
NS = 16
TILES = NC * NS
CH = 80
NPAD = 10112
RPT = NPAD // NS
NBUF = 4

BE = 6400
CBS = 800
WN = 64
BN = 2000


def _make_msg_body(nbe_h):
  def _msg_body(x_ref, w_ref, b_ref, dv_ref, ds_ref, o_ref, cnt_out, cnt_ref):
    pid = pl.program_id(0)

    @pl.when(pid == 0)
    def _init():
        cnt_ref[...] = jnp.zeros((NPAD, D), jnp.float32)

    acc = lax.dot_general(
        x_ref[...], w_ref[...], (((1,), (1,)), ((), ())),
        preferred_element_type=jnp.float32)
    o_ref[...] = jnp.maximum(acc + b_ref[...], 0.0)

    for j in range(BE // CBS):
        d0 = ds_ref[0, 0, j * CBS]
        dl = ds_ref[0, 0, j * CBS + CBS - 1]
        base = (d0 // 8) * 8
        span_ok = dl - base < WN

        @pl.when(span_ok)
        def _windowed(j=j, base=base):
            dv = dv_ref[0, 0, pl.ds(j * CBS, CBS)]
            ids = lax.broadcasted_iota(jnp.int32, (WN, CBS), 0) + base
            oh = (ids == dv[None, :]).astype(jnp.float32)
            colcnt = jnp.sum(oh, axis=1, keepdims=True)
            cnt_ref[pl.ds(base, WN), :] += colcnt

        @pl.when(jnp.logical_not(span_ok))
        def _fallback(j=j):
            def fb(e, carry):
                de = ds_ref[0, 0, e]
                cnt_ref[pl.ds(de, 1), :] += 1.0
                return carry
            lax.fori_loop(j * CBS, (j + 1) * CBS, fb, 0)

    @pl.when(pid == nbe_h - 1)
    def _finish():
        cnt_out[...] = cnt_ref[...]

  return _msg_body


def _compute_msg(src_emb, W, b2, dst_r, h):
    e_off, split_e = SPLITS[h]
    off = e_off // BE
    nbe_h = split_e // BE
    return pl.pallas_call(
        _make_msg_body(nbe_h),
        grid=(nbe_h,),
        in_specs=[
            pl.BlockSpec((BE, D), lambda i: (i + off, 0)),
            pl.BlockSpec((D, D), lambda i: (0, 0)),
            pl.BlockSpec((1, D), lambda i: (0, 0)),
            pl.BlockSpec((1, 1, BE), lambda i: (i + off, 0, 0)),
            pl.BlockSpec((1, 1, BE), lambda i: (i + off, 0, 0),
                         memory_space=pltpu.SMEM),
        ],
        out_specs=[
            pl.BlockSpec((BE, D), lambda i: (i, 0)),
            pl.BlockSpec((NPAD, D), lambda i: (0, 0)),
        ],
        out_shape=[
            jax.ShapeDtypeStruct((split_e, D), jnp.float32),
            jax.ShapeDtypeStruct((NPAD, D), jnp.float32),
        ],
        scratch_shapes=[pltpu.VMEM((NPAD, D), jnp.float32)],
    )(src_emb, W, b2, dst_r, dst_r)


def _make_scatter_body(dst_off, ept):
    n80 = ept // CH
    rem = ept % CH

    def _scatter_body(msg_hbm, dst_hbm, prev_hbm, acc_out,
                      idx0, idx1, idx2, idx3, idxt,
                      buf0, buf1, buf2, buf3, acc_sh,
                      gs0, gs1, gs2, gs3, is0, is1, is2, is3,
                      ss0, ss1, ss2, ss3):
        c = lax.axis_index("c")
        s = lax.axis_index("s")
        tid = c * NS + s
        r0 = pl.multiple_of(s * RPT, 8)
        pltpu.sync_copy(prev_hbm.at[c, pl.ds(r0, RPT)],
                        acc_sh.at[pl.ds(r0, RPT)])
        plsc.subcore_barrier()

        base = tid * ept
        idx = (idx0, idx1, idx2, idx3)
        buf = (buf0, buf1, buf2, buf3)
        gs = (gs0, gs1, gs2, gs3)
        isem = (is0, is1, is2, is3)
        ss = (ss0, ss1, ss2, ss3)

        def start_gather(i, b):
            e0 = pl.multiple_of(base + i * CH, 8)
            ed = pl.multiple_of(dst_off + base + i * CH, 8)
            pltpu.async_copy(msg_hbm.at[pl.ds(e0, CH)], buf[b], gs[b])
            pltpu.async_copy(dst_hbm.at[pl.ds(ed, CH)], idx[b].at[0], isem[b])

        def wait_gather(b):
            pltpu.make_async_copy(msg_hbm.at[pl.ds(0, CH)], buf[b],
                                  gs[b]).wait()
            pltpu.make_async_copy(dst_hbm.at[pl.ds(0, CH)], idx[b].at[0],
                                  isem[b]).wait()

        def start_scatter(b):
            pltpu.async_copy(buf[b], acc_sh.at[idx[b].at[0]], ss[b], add=True)

        def wait_scatter(b):
            pltpu.make_async_copy(msg_hbm.at[pl.ds(0, CH)], buf[b],
                                  ss[b]).wait()

        start_gather(0, 0)
        for i in range(min(3, n80)):
            if i + 1 < n80:
                start_gather(i + 1, (i + 1) % NBUF)
            wait_gather(i % NBUF)
            start_scatter(i % NBUF)

        def visit(i, b):
            wait_scatter((b + 1) % NBUF)
            start_gather(i + 1, (b + 1) % NBUF)
            wait_gather(b)
            start_scatter(b)

        def body(p, carry):
            i0 = 3 + 4 * p
            visit(i0, 3)
            visit(i0 + 1, 0)
            visit(i0 + 2, 1)
            visit(i0 + 3, 2)
            return carry

        ngrp = max(0, n80 - 5) // NBUF
        lax.fori_loop(0, ngrp, body, 0)
        for i in range(3 + 4 * ngrp, n80):
            b = i % NBUF
            wait_scatter((b + 1) % NBUF)
            if i + 1 < n80:
                start_gather(i + 1, (b + 1) % NBUF)
            wait_gather(b)
            start_scatter(b)
        for i in range(max(0, n80 - 3), n80):
            wait_scatter(i % NBUF)

        if rem:
            e0 = pl.multiple_of(base + n80 * CH, 8)
            ed = pl.multiple_of(dst_off + base + n80 * CH, 8)
            pltpu.sync_copy(dst_hbm.at[pl.ds(ed, rem)], idxt.at[0])
            pltpu.sync_copy(msg_hbm.at[pl.ds(e0, rem)],
                            buf0.at[pl.ds(0, rem)])
            pltpu.sync_copy(buf0.at[pl.ds(0, rem)], acc_sh.at[idxt.at[0]],
                            add=True)

        plsc.subcore_barrier()
        pltpu.sync_copy(acc_sh.at[pl.ds(r0, RPT)],
                        acc_out.at[c, pl.ds(r0, RPT)])

    return _scatter_body


@functools.cache
def _get_scatter(h):
    e_off, split_e = SPLITS[h]
    rem = (split_e // TILES) % CH
    return pl.kernel(
        _make_scatter_body(e_off, split_e // TILES),
        out_type=jax.ShapeDtypeStruct((NC, NPAD, D), jnp.float32),
        mesh=plsc.VectorSubcoreMesh(core_axis_name="c", subcore_axis_name="s"),
        scratch_types=(
            tuple(pltpu.VMEM((1, CH), jnp.int32) for _ in range(NBUF))
            + (pltpu.VMEM((1, max(rem, 8)), jnp.int32),)
            + tuple(pltpu.VMEM((CH, D), jnp.float32) for _ in range(NBUF))
            + (pltpu.VMEM_SHARED((NPAD, D), jnp.float32),)
            + tuple(pltpu.SemaphoreType.DMA for _ in range(3 * NBUF))
        ),
    )


def _combine_body(a_ref, c0_ref, c1_ref, c2_ref, c3_ref, o_ref):
    a = a_ref[0] + a_ref[1]
    cnt = c0_ref[...] + c1_ref[...] + c2_ref[...] + c3_ref[...]
    o_ref[...] = a / jnp.maximum(cnt, 1.0)


def _combine(acc, cnts):
    return pl.pallas_call(
        _combine_body,
        grid=(N_NODES // BN,),
        in_specs=[pl.BlockSpec((NC, BN, D), lambda i: (0, i, 0))]
        + [pl.BlockSpec((BN, D), lambda i: (i, 0)) for _ in range(NSPLIT)],
        out_specs=pl.BlockSpec((BN, D), lambda i: (i, 0)),
        out_shape=jax.ShapeDtypeStruct((N_NODES, D), jnp.float32),
    )(acc, *cnts)


def kernel(src_emb, src_emb_in, dst, W, b):
    del src_emb_in
    dst_i = dst.astype(jnp.int32)
    b2 = b.reshape(1, D)
    dst_r = dst_i.reshape(N_EDGES // BE, 1, BE)

    acc = jnp.zeros((NC, NPAD, D), jnp.float32)
    cnts = []
    for h in range(NSPLIT):
        msg_h, cnt_h = _compute_msg(src_emb, W, b2, dst_r, h)
        acc = _get_scatter(h)(msg_h, dst_i, acc)
        cnts.append(cnt_h)
    return _combine(acc, cnts)


# speedup vs baseline: 1.0812x; 1.0812x over previous
import functools

import jax
import jax.numpy as jnp
from jax import lax
from jax.experimental import pallas as pl
from jax.experimental.pallas import tpu as pltpu
from jax.experimental.pallas import tpu_sc as plsc

N_NODES = 10000
N_EDGES = 320000
D = 128

NSPLIT = 2
SPLIT_E = N_EDGES // NSPLIT

NC = 2
NS = 16
TILES = NC * NS
EPT = SPLIT_E // TILES
CH = 80
NPAD = 10112
RPT = NPAD // NS
NBUF = 4

BE = 8000
NBE_H = SPLIT_E // BE
CBS = 1000
WN = 64
BN = 2000


def _msg_body(x_ref, w_ref, b_ref, dv_ref, ds_ref, o_ref, cnt_out, cnt_ref):
    pid = pl.program_id(0)

    @pl.when(pid == 0)
    def _init():
        cnt_ref[...] = jnp.zeros((NPAD, D), jnp.float32)

    acc = lax.dot_general(
        x_ref[...], w_ref[...], (((1,), (1,)), ((), ())),
        preferred_element_type=jnp.float32)
    o_ref[...] = jnp.maximum(acc + b_ref[...], 0.0)

    for j in range(BE // CBS):
        d0 = ds_ref[0, 0, j * CBS]
        dl = ds_ref[0, 0, j * CBS + CBS - 1]
        base = (d0 // 8) * 8
        span_ok = dl - base < WN

        @pl.when(span_ok)
        def _windowed(j=j, base=base):
            dv = dv_ref[0, 0, pl.ds(j * CBS, CBS)]
            ids = lax.broadcasted_iota(jnp.int32, (WN, CBS), 0) + base
            oh = (ids == dv[None, :]).astype(jnp.float32)
            colcnt = jnp.sum(oh, axis=1, keepdims=True)
            cnt_ref[pl.ds(base, WN), :] += colcnt

        @pl.when(jnp.logical_not(span_ok))
        def _fallback(j=j):
            def fb(e, carry):
                de = ds_ref[0, 0, e]
                cnt_ref[pl.ds(de, 1), :] += 1.0
                return carry
            lax.fori_loop(j * CBS, (j + 1) * CBS, fb, 0)

    @pl.when(pid == NBE_H - 1)
    def _finish():
        cnt_out[...] = cnt_ref[...]


def _compute_msg(src_emb, W, b2, dst_r, h):
    off = h * NBE_H
    return pl.pallas_call(
        _msg_body,
        grid=(NBE_H,),
        in_specs=[
            pl.BlockSpec((BE, D), lambda i: (i + off, 0)),
            pl.BlockSpec((D, D), lambda i: (0, 0)),
            pl.BlockSpec((1, D), lambda i: (0, 0)),
            pl.BlockSpec((1, 1, BE), lambda i: (i + off, 0, 0)),
            pl.BlockSpec((1, 1, BE), lambda i: (i + off, 0, 0),
                         memory_space=pltpu.SMEM),
        ],
        out_specs=[
            pl.BlockSpec((BE, D), lambda i: (i, 0)),
            pl.BlockSpec((NPAD, D), lambda i: (0, 0)),
        ],
        out_shape=[
            jax.ShapeDtypeStruct((SPLIT_E, D), jnp.float32),
            jax.ShapeDtypeStruct((NPAD, D), jnp.float32),
        ],
        scratch_shapes=[pltpu.VMEM((NPAD, D), jnp.float32)],
    )(src_emb, W, b2, dst_r, dst_r)


def _make_scatter_body(dst_off):
    n80 = EPT // CH
    rem = EPT % CH

    def _scatter_body(msg_hbm, dst_hbm, zacc_hbm, acc_out,
                      idx0, idx1, idx2, idx3, idxt,
                      buf0, buf1, buf2, buf3, acc_sh,
                      gs0, gs1, gs2, gs3, is0, is1, is2, is3,
                      ss0, ss1, ss2, ss3):
        c = lax.axis_index("c")
        s = lax.axis_index("s")
        tid = c * NS + s
        r0 = pl.multiple_of(s * RPT, 8)
        pltpu.sync_copy(zacc_hbm.at[pl.ds(r0, RPT)], acc_sh.at[pl.ds(r0, RPT)])
        plsc.subcore_barrier()

        base = tid * EPT
        idx = (idx0, idx1, idx2, idx3)
        buf = (buf0, buf1, buf2, buf3)
        gs = (gs0, gs1, gs2, gs3)
        isem = (is0, is1, is2, is3)
        ss = (ss0, ss1, ss2, ss3)

        def start_gather(i, b):
            e0 = pl.multiple_of(base + i * CH, 8)
            ed = pl.multiple_of(dst_off + base + i * CH, 8)
            pltpu.async_copy(msg_hbm.at[pl.ds(e0, CH)], buf[b], gs[b])
            pltpu.async_copy(dst_hbm.at[pl.ds(ed, CH)], idx[b].at[0], isem[b])

        def wait_gather(b):
            pltpu.make_async_copy(msg_hbm.at[pl.ds(0, CH)], buf[b],
                                  gs[b]).wait()
            pltpu.make_async_copy(dst_hbm.at[pl.ds(0, CH)], idx[b].at[0],
                                  isem[b]).wait()

        def start_scatter(b):
            pltpu.async_copy(buf[b], acc_sh.at[idx[b].at[0]], ss[b], add=True)

        def wait_scatter(b):
            pltpu.make_async_copy(msg_hbm.at[pl.ds(0, CH)], buf[b],
                                  ss[b]).wait()

        start_gather(0, 0)
        for i in range(min(3, n80)):
            if i + 1 < n80:
                start_gather(i + 1, (i + 1) % NBUF)
            wait_gather(i % NBUF)
            start_scatter(i % NBUF)

        def visit(i, b):
            wait_scatter((b + 1) % NBUF)
            start_gather(i + 1, (b + 1) % NBUF)
            wait_gather(b)
            start_scatter(b)

        def body(p, carry):
            i0 = 3 + 4 * p
            visit(i0, 3)
            visit(i0 + 1, 0)
            visit(i0 + 2, 1)
            visit(i0 + 3, 2)
            return carry

        ngrp = max(0, n80 - 5) // NBUF
        lax.fori_loop(0, ngrp, body, 0)
        for i in range(3 + 4 * ngrp, n80):
            b = i % NBUF
            wait_scatter((b + 1) % NBUF)
            if i + 1 < n80:
                start_gather(i + 1, (b + 1) % NBUF)
            wait_gather(b)
            start_scatter(b)
        for i in range(max(0, n80 - 3), n80):
            wait_scatter(i % NBUF)

        if rem:
            e0 = pl.multiple_of(base + n80 * CH, 8)
            ed = pl.multiple_of(dst_off + base + n80 * CH, 8)
            pltpu.sync_copy(dst_hbm.at[pl.ds(ed, rem)], idxt.at[0])
            pltpu.sync_copy(msg_hbm.at[pl.ds(e0, rem)],
                            buf0.at[pl.ds(0, rem)])
            pltpu.sync_copy(buf0.at[pl.ds(0, rem)], acc_sh.at[idxt.at[0]],
                            add=True)

        plsc.subcore_barrier()
        pltpu.sync_copy(acc_sh.at[pl.ds(r0, RPT)],
                        acc_out.at[c, pl.ds(r0, RPT)])

    return _scatter_body


@functools.cache
def _get_scatter(h):
    rem = EPT % CH
    return pl.kernel(
        _make_scatter_body(h * SPLIT_E),
        out_type=jax.ShapeDtypeStruct((NC, NPAD, D), jnp.float32),
        mesh=plsc.VectorSubcoreMesh(core_axis_name="c", subcore_axis_name="s"),
        scratch_types=(
            tuple(pltpu.VMEM((1, CH), jnp.int32) for _ in range(NBUF))
            + (pltpu.VMEM((1, max(rem, 8)), jnp.int32),)
            + tuple(pltpu.VMEM((CH, D), jnp.float32) for _ in range(NBUF))
            + (pltpu.VMEM_SHARED((NPAD, D), jnp.float32),)
            + tuple(pltpu.SemaphoreType.DMA for _ in range(3 * NBUF))
        ),
    )


def _combine_body(a0_ref, a1_ref, c0_ref, c1_ref, o_ref):
    a = a0_ref[0] + a0_ref[1] + a1_ref[0] + a1_ref[1]
    cnt = c0_ref[...] + c1_ref[...]
    o_ref[...] = a / jnp.maximum(cnt, 1.0)


def _combine(acc_a, acc_b, cnt_a, cnt_b):
    return pl.pallas_call(
        _combine_body,
        grid=(N_NODES // BN,),
        in_specs=[
            pl.BlockSpec((NC, BN, D), lambda i: (0, i, 0)),
            pl.BlockSpec((NC, BN, D), lambda i: (0, i, 0)),
            pl.BlockSpec((BN, D), lambda i: (i, 0)),
            pl.BlockSpec((BN, D), lambda i: (i, 0)),
        ],
        out_specs=pl.BlockSpec((BN, D), lambda i: (i, 0)),
        out_shape=jax.ShapeDtypeStruct((N_NODES, D), jnp.float32),
    )(acc_a, acc_b, cnt_a, cnt_b)


def kernel(src_emb, src_emb_in, dst, W, b):
    del src_emb_in
    dst_i = dst.astype(jnp.int32)
    b2 = b.reshape(1, D)
    dst_r = dst_i.reshape(N_EDGES // BE, 1, BE)
    zacc = jnp.zeros((NPAD, D), jnp.float32)

    msg_a, cnt_a = _compute_msg(src_emb, W, b2, dst_r, 0)
    acc_a = _get_scatter(0)(msg_a, dst_i, zacc)
    msg_b, cnt_b = _compute_msg(src_emb, W, b2, dst_r, 1)
    acc_b = _get_scatter(1)(msg_b, dst_i, zacc)
    return _combine(acc_a, acc_b, cnt_a, cnt_b)
